# baseline (speedup 1.0000x reference)
import functools

import jax
import jax.numpy as jnp
from jax import lax
from jax.experimental import pallas as pl
from jax.experimental.pallas import tpu as pltpu
from jax.experimental.pallas import tpu_sc as plsc

N_OBS = 100000
N_STATES = 128
BATCH = 1024
SEQ = 200

_B = BATCH * SEQ
_NC = 2
_NS = 16
_NW = _NC * _NS
_PER_W = _B // _NW
_CHUNK = 128
_N_CHUNKS = _PER_W // _CHUNK
_NBUF = 4
_PREF = 2

_LOG_ROWS = 1000


def _log_body(p_ref, o_ref):
    o_ref[...] = jnp.log(p_ref[...])


def _table_log(probs):
    return pl.pallas_call(
        _log_body,
        out_shape=jax.ShapeDtypeStruct((N_OBS, N_STATES), jnp.float32),
        grid=(N_OBS // _LOG_ROWS,),
        in_specs=[pl.BlockSpec((_LOG_ROWS, N_STATES), lambda i: (i, 0))],
        out_specs=pl.BlockSpec((_LOG_ROWS, N_STATES), lambda i: (i, 0)),
    )(probs)


@functools.partial(
    pl.kernel,
    out_type=jax.ShapeDtypeStruct((_B, N_STATES), jnp.float32),
    mesh=plsc.VectorSubcoreMesh(core_axis_name="c", subcore_axis_name="s"),
    scratch_types=[
        pltpu.VMEM((_N_CHUNKS, _CHUNK), jnp.int32),
        pltpu.VMEM((_NBUF, _CHUNK, N_STATES), jnp.float32),
        pltpu.SemaphoreType.DMA((_NBUF,)),
        pltpu.SemaphoreType.DMA((_NBUF,)),
    ],
)
def _sc_gather(x_hbm, lp_hbm, out_hbm, idx_v, buf, gsem, ssem):
    wid = lax.axis_index("s") * _NC + lax.axis_index("c")
    base_w = wid * _PER_W

    pltpu.sync_copy(x_hbm.at[wid], idx_v)

    def start_gather(g, p):
        pltpu.async_copy(lp_hbm.at[idx_v.at[g]], buf.at[p], gsem.at[p])

    def wait_gather(p):
        pltpu.make_async_copy(lp_hbm.at[idx_v.at[0]], buf.at[p],
                              gsem.at[p]).wait()

    def out_slice(g):
        return out_hbm.at[pl.ds(base_w + g * _CHUNK, _CHUNK)]

    def start_scatter(g, p):
        pltpu.async_copy(buf.at[p], out_slice(g), ssem.at[p])

    def wait_scatter(g, p):
        pltpu.make_async_copy(buf.at[p], out_slice(g), ssem.at[p]).wait()

    for g in range(_PREF):
        start_gather(g, g % _NBUF)

    def pipe_body(i, carry):
        for j in range(_NBUF):
            g = _NBUF * i + j
            p = j

            @pl.when(g + _PREF < _N_CHUNKS)
            def _():
                @pl.when(g + _PREF >= _NBUF)
                def _():
                    wait_scatter(g + _PREF - _NBUF, (g + _PREF) % _NBUF)
                start_gather(g + _PREF, (g + _PREF) % _NBUF)

            wait_gather(p)
            start_scatter(g, p)
        return carry

    lax.fori_loop(0, _N_CHUNKS // _NBUF, pipe_body, 0)
    for g in range(_N_CHUNKS - (_N_CHUNKS % _NBUF), _N_CHUNKS):
        p = g % _NBUF
        wait_gather(p)
        start_scatter(g, p)
    for g in range(_N_CHUNKS - _NBUF, _N_CHUNKS):
        wait_scatter(g, g % _NBUF)


def kernel(x, probs):
    xf = x.reshape(_NW, _N_CHUNKS, _CHUNK).astype(jnp.int32)
    lp = _table_log(probs)
    out = _sc_gather(xf, lp)
    return out.reshape(BATCH, SEQ, N_STATES)
